# baseline (device time: 197571 ns/iter reference)
import jax
import jax.numpy as jnp
from jax import lax
from jax.experimental import pallas as pl
from jax.experimental.pallas import tpu as pltpu

N_DEV = 8


def kernel(x, w_mat):
    m_per, k = x.shape
    _, n_per = w_mat.shape
    half = m_per // 2

    def body(x_ref, w_ref, out_ref, buf_a, buf_b, w_bf,
             send_a, recv_a, send_b, recv_b):
        my = lax.axis_index("i")
        left = (my + N_DEV - 1) % N_DEV
        right = (my + 1) % N_DEV

        barrier_sem = pltpu.get_barrier_semaphore()
        for nbr in (left, right):
            pl.semaphore_signal(
                barrier_sem, inc=1,
                device_id=(nbr,), device_id_type=pl.DeviceIdType.MESH,
            )
        pl.semaphore_wait(barrier_sem, 2)

        w_bf[...] = w_ref[...].astype(jnp.bfloat16)
        buf_a[0] = x_ref[:half, :].astype(jnp.bfloat16)
        buf_b[0] = x_ref[half:, :].astype(jnp.bfloat16)

        own = jnp.dot(
            jnp.concatenate([buf_a[0], buf_b[0]], axis=0), w_bf[...],
            preferred_element_type=jnp.float32,
        )
        out_ref[pl.ds(my * m_per, m_per), :] = jnp.maximum(own, 0.0)

        for h in range(N_DEV - 1):
            s = h % 2
            r = (h + 1) % 2
            rdma_a = pltpu.make_async_remote_copy(
                src_ref=buf_a.at[s],
                dst_ref=buf_a.at[r],
                send_sem=send_a.at[h],
                recv_sem=recv_a.at[h],
                device_id=(right,),
                device_id_type=pl.DeviceIdType.MESH,
            )
            rdma_b = pltpu.make_async_remote_copy(
                src_ref=buf_b.at[s],
                dst_ref=buf_b.at[r],
                send_sem=send_b.at[h],
                recv_sem=recv_b.at[h],
                device_id=(left,),
                device_id_type=pl.DeviceIdType.MESH,
            )
            rdma_a.start()
            rdma_b.start()
            rdma_a.wait()
            rdma_b.wait()

            origin_a = (my - h - 1) % N_DEV
            origin_b = (my + h + 1) % N_DEV
            ya = jnp.dot(buf_a[r], w_bf[...],
                         preferred_element_type=jnp.float32)
            out_ref[pl.ds(origin_a * m_per, half), :] = jnp.maximum(ya, 0.0)
            yb = jnp.dot(buf_b[r], w_bf[...],
                         preferred_element_type=jnp.float32)
            out_ref[pl.ds(origin_b * m_per + half, half), :] = (
                jnp.maximum(yb, 0.0))

    return pl.pallas_call(
        body,
        out_shape=jax.ShapeDtypeStruct((N_DEV * m_per, n_per), jnp.float32),
        in_specs=[
            pl.BlockSpec(memory_space=pltpu.VMEM),
            pl.BlockSpec(memory_space=pltpu.VMEM),
        ],
        out_specs=pl.BlockSpec(memory_space=pltpu.VMEM),
        scratch_shapes=[
            pltpu.VMEM((2, half, k), jnp.bfloat16),
            pltpu.VMEM((2, half, k), jnp.bfloat16),
            pltpu.VMEM((k, n_per), jnp.bfloat16),
            pltpu.SemaphoreType.DMA((N_DEV - 1,)),
            pltpu.SemaphoreType.DMA((N_DEV - 1,)),
            pltpu.SemaphoreType.DMA((N_DEV - 1,)),
            pltpu.SemaphoreType.DMA((N_DEV - 1,)),
        ],
        compiler_params=pltpu.CompilerParams(collective_id=0),
    )(x, w_mat)


# device time: 186329 ns/iter; 1.0603x vs baseline; 1.0603x over previous
import jax
import jax.numpy as jnp
from jax import lax
from jax.experimental import pallas as pl
from jax.experimental.pallas import tpu as pltpu

N_DEV = 8
DEPTH = 3


def kernel(x, w_mat):
    m_per, k = x.shape
    _, n_per = w_mat.shape
    half = m_per // 2

    def body(x_ref, w_ref, out_ref, buf_a, buf_b, w_bf,
             send_a, recv_a, send_b, recv_b, credit_a, credit_b):
        my = lax.axis_index("i")
        left = (my + N_DEV - 1) % N_DEV
        right = (my + 1) % N_DEV

        barrier_sem = pltpu.get_barrier_semaphore()
        for nbr in (left, right):
            pl.semaphore_signal(
                barrier_sem, inc=1,
                device_id=(nbr,), device_id_type=pl.DeviceIdType.MESH,
            )
        pl.semaphore_wait(barrier_sem, 2)

        w_bf[...] = w_ref[...].astype(jnp.bfloat16)
        buf_a[0] = x_ref[:half, :].astype(jnp.bfloat16)
        buf_b[0] = x_ref[half:, :].astype(jnp.bfloat16)

        def gemm_store(h):
            s = h % DEPTH
            origin_a = (my - h) % N_DEV
            origin_b = (my + h) % N_DEV
            ya = jnp.dot(buf_a[s], w_bf[...],
                         preferred_element_type=jnp.float32)
            out_ref[pl.ds(origin_a * m_per, half), :] = jnp.maximum(ya, 0.0)
            yb = jnp.dot(buf_b[s], w_bf[...],
                         preferred_element_type=jnp.float32)
            out_ref[pl.ds(origin_b * m_per + half, half), :] = (
                jnp.maximum(yb, 0.0))

        for h in range(N_DEV - 1):
            s = h % DEPTH
            r = (h + 1) % DEPTH
            if h >= 2:
                pl.semaphore_wait(credit_a, 1)
                pl.semaphore_wait(credit_b, 1)
            rdma_a = pltpu.make_async_remote_copy(
                src_ref=buf_a.at[s],
                dst_ref=buf_a.at[r],
                send_sem=send_a.at[h],
                recv_sem=recv_a.at[h],
                device_id=(right,),
                device_id_type=pl.DeviceIdType.MESH,
            )
            rdma_b = pltpu.make_async_remote_copy(
                src_ref=buf_b.at[s],
                dst_ref=buf_b.at[r],
                send_sem=send_b.at[h],
                recv_sem=recv_b.at[h],
                device_id=(left,),
                device_id_type=pl.DeviceIdType.MESH,
            )
            rdma_a.start()
            rdma_b.start()

            gemm_store(h)

            rdma_a.wait_send()
            rdma_b.wait_send()
            if h <= 4:
                pl.semaphore_signal(
                    credit_a, inc=1,
                    device_id=(left,), device_id_type=pl.DeviceIdType.MESH,
                )
                pl.semaphore_signal(
                    credit_b, inc=1,
                    device_id=(right,), device_id_type=pl.DeviceIdType.MESH,
                )
            rdma_a.wait_recv()
            rdma_b.wait_recv()

        gemm_store(N_DEV - 1)

    return pl.pallas_call(
        body,
        out_shape=jax.ShapeDtypeStruct((N_DEV * m_per, n_per), jnp.float32),
        in_specs=[
            pl.BlockSpec(memory_space=pltpu.VMEM),
            pl.BlockSpec(memory_space=pltpu.VMEM),
        ],
        out_specs=pl.BlockSpec(memory_space=pltpu.VMEM),
        scratch_shapes=[
            pltpu.VMEM((DEPTH, half, k), jnp.bfloat16),
            pltpu.VMEM((DEPTH, half, k), jnp.bfloat16),
            pltpu.VMEM((k, n_per), jnp.bfloat16),
            pltpu.SemaphoreType.DMA((N_DEV - 1,)),
            pltpu.SemaphoreType.DMA((N_DEV - 1,)),
            pltpu.SemaphoreType.DMA((N_DEV - 1,)),
            pltpu.SemaphoreType.DMA((N_DEV - 1,)),
            pltpu.SemaphoreType.REGULAR,
            pltpu.SemaphoreType.REGULAR,
        ],
        compiler_params=pltpu.CompilerParams(collective_id=0),
    )(x, w_mat)


# device time: 173161 ns/iter; 1.1410x vs baseline; 1.0760x over previous
import jax
import jax.numpy as jnp
from jax import lax
from jax.experimental import pallas as pl
from jax.experimental.pallas import tpu as pltpu

N_DEV = 8
DEPTH = 3
SUB = 2


def kernel(x, w_mat):
    m_per, k = x.shape
    _, n_per = w_mat.shape
    half = m_per // 2
    sub = half // SUB

    def body(x_ref, w_ref, out_ref, buf_a, buf_b, w_bf,
             send_a, recv_a, send_b, recv_b, credit_a, credit_b):
        my = lax.axis_index("i")
        left = (my + N_DEV - 1) % N_DEV
        right = (my + 1) % N_DEV

        barrier_sem = pltpu.get_barrier_semaphore()
        for nbr in (left, right):
            pl.semaphore_signal(
                barrier_sem, inc=1,
                device_id=(nbr,), device_id_type=pl.DeviceIdType.MESH,
            )
        pl.semaphore_wait(barrier_sem, 2)

        def rdma(buf, send_sems, recv_sems, j, h, dev):
            s = h % DEPTH
            r = (h + 1) % DEPTH
            return pltpu.make_async_remote_copy(
                src_ref=buf.at[s, pl.ds(j * sub, sub), :],
                dst_ref=buf.at[r, pl.ds(j * sub, sub), :],
                send_sem=send_sems.at[j, h],
                recv_sem=recv_sems.at[j, h],
                device_id=(dev,),
                device_id_type=pl.DeviceIdType.MESH,
            )

        def rdma_a(j, h):
            return rdma(buf_a, send_a, recv_a, j, h, right)

        def rdma_b(j, h):
            return rdma(buf_b, send_b, recv_b, j, h, left)

        def gemm_store(h):
            s = h % DEPTH
            origin_a = (my - h) % N_DEV
            origin_b = (my + h) % N_DEV
            ya = jnp.dot(buf_a[s], w_bf[...],
                         preferred_element_type=jnp.float32)
            out_ref[pl.ds(origin_a * m_per, half), :] = jnp.maximum(ya, 0.0)
            yb = jnp.dot(buf_b[s], w_bf[...],
                         preferred_element_type=jnp.float32)
            out_ref[pl.ds(origin_b * m_per + half, half), :] = (
                jnp.maximum(yb, 0.0))

        buf_a[0, :sub, :] = x_ref[:sub, :].astype(jnp.bfloat16)
        rdma_a(0, 0).start()
        buf_b[0, :sub, :] = x_ref[half:half + sub, :].astype(jnp.bfloat16)
        rdma_b(0, 0).start()
        buf_a[0, sub:, :] = x_ref[sub:half, :].astype(jnp.bfloat16)
        rdma_a(1, 0).start()
        buf_b[0, sub:, :] = x_ref[half + sub:, :].astype(jnp.bfloat16)
        rdma_b(1, 0).start()
        w_bf[...] = w_ref[...].astype(jnp.bfloat16)
        gemm_store(0)

        for h in range(1, N_DEV - 1):
            if h >= 2:
                pl.semaphore_wait(credit_a, 1)
                pl.semaphore_wait(credit_b, 1)
            rdma_a(0, h - 1).wait_recv()
            rdma_a(0, h).start()
            rdma_b(0, h - 1).wait_recv()
            rdma_b(0, h).start()
            rdma_a(1, h - 1).wait_recv()
            rdma_a(1, h).start()
            rdma_b(1, h - 1).wait_recv()
            rdma_b(1, h).start()
            for j in range(SUB):
                rdma_a(j, h - 1).wait_send()
                rdma_b(j, h - 1).wait_send()
            gemm_store(h)
            if h <= 5:
                pl.semaphore_signal(
                    credit_a, inc=1,
                    device_id=(left,), device_id_type=pl.DeviceIdType.MESH,
                )
                pl.semaphore_signal(
                    credit_b, inc=1,
                    device_id=(right,), device_id_type=pl.DeviceIdType.MESH,
                )

        for j in range(SUB):
            rdma_a(j, N_DEV - 2).wait_recv()
            rdma_b(j, N_DEV - 2).wait_recv()
        for j in range(SUB):
            rdma_a(j, N_DEV - 2).wait_send()
            rdma_b(j, N_DEV - 2).wait_send()
        gemm_store(N_DEV - 1)

    return pl.pallas_call(
        body,
        out_shape=jax.ShapeDtypeStruct((N_DEV * m_per, n_per), jnp.float32),
        in_specs=[
            pl.BlockSpec(memory_space=pltpu.VMEM),
            pl.BlockSpec(memory_space=pltpu.VMEM),
        ],
        out_specs=pl.BlockSpec(memory_space=pltpu.VMEM),
        scratch_shapes=[
            pltpu.VMEM((DEPTH, half, k), jnp.bfloat16),
            pltpu.VMEM((DEPTH, half, k), jnp.bfloat16),
            pltpu.VMEM((k, n_per), jnp.bfloat16),
            pltpu.SemaphoreType.DMA((SUB, N_DEV - 1)),
            pltpu.SemaphoreType.DMA((SUB, N_DEV - 1)),
            pltpu.SemaphoreType.DMA((SUB, N_DEV - 1)),
            pltpu.SemaphoreType.DMA((SUB, N_DEV - 1)),
            pltpu.SemaphoreType.REGULAR,
            pltpu.SemaphoreType.REGULAR,
        ],
        compiler_params=pltpu.CompilerParams(collective_id=0),
    )(x, w_mat)


# device time: 172591 ns/iter; 1.1447x vs baseline; 1.0033x over previous
import jax
import jax.numpy as jnp
from jax import lax
from jax.experimental import pallas as pl
from jax.experimental.pallas import tpu as pltpu

N_DEV = 8
DEPTH = 3
SUB = 2


def kernel(x, w_mat):
    m_per, k = x.shape
    _, n_per = w_mat.shape
    half = m_per // 2
    sub = half // SUB

    def body(x_ref, w_ref, out_ref, buf_a, buf_b, w_bf,
             send_a, recv_a, send_b, recv_b, credit_a, credit_b):
        my = lax.axis_index("i")
        left = (my + N_DEV - 1) % N_DEV
        right = (my + 1) % N_DEV

        barrier_sem = pltpu.get_barrier_semaphore()
        for nbr in (left, right):
            pl.semaphore_signal(
                barrier_sem, inc=1,
                device_id=(nbr,), device_id_type=pl.DeviceIdType.MESH,
            )
        pl.semaphore_wait(barrier_sem, 2)

        def rdma(buf, send_sems, recv_sems, j, h, dev):
            s = h % DEPTH
            r = (h + 1) % DEPTH
            return pltpu.make_async_remote_copy(
                src_ref=buf.at[s, pl.ds(j * sub, sub), :],
                dst_ref=buf.at[r, pl.ds(j * sub, sub), :],
                send_sem=send_sems.at[j, h],
                recv_sem=recv_sems.at[j, h],
                device_id=(dev,),
                device_id_type=pl.DeviceIdType.MESH,
            )

        def rdma_a(j, h):
            return rdma(buf_a, send_a, recv_a, j, h, right)

        def rdma_b(j, h):
            return rdma(buf_b, send_b, recv_b, j, h, left)

        def gemm_store(h):
            s = h % DEPTH
            origin_a = (my - h) % N_DEV
            origin_b = (my + h) % N_DEV
            ya = jnp.dot(buf_a[s], w_bf[...],
                         preferred_element_type=jnp.float32)
            out_ref[pl.ds(origin_a * m_per, half), :] = jnp.maximum(ya, 0.0)
            yb = jnp.dot(buf_b[s], w_bf[...],
                         preferred_element_type=jnp.float32)
            out_ref[pl.ds(origin_b * m_per + half, half), :] = (
                jnp.maximum(yb, 0.0))

        buf_a[0, :sub, :] = x_ref[:sub, :].astype(jnp.bfloat16)
        rdma_a(0, 0).start()
        buf_b[0, :sub, :] = x_ref[half:half + sub, :].astype(jnp.bfloat16)
        rdma_b(0, 0).start()
        buf_a[0, sub:, :] = x_ref[sub:half, :].astype(jnp.bfloat16)
        rdma_a(1, 0).start()
        buf_b[0, sub:, :] = x_ref[half + sub:, :].astype(jnp.bfloat16)
        rdma_b(1, 0).start()
        w_bf[...] = w_ref[...].astype(jnp.bfloat16)
        gemm_store(0)

        for h in range(1, N_DEV - 1):
            if h >= 2:
                pl.semaphore_wait(credit_a, 1)
                pl.semaphore_wait(credit_b, 1)
            rdma_a(0, h - 1).wait_recv()
            rdma_a(0, h).start()
            rdma_b(0, h - 1).wait_recv()
            rdma_b(0, h).start()
            rdma_a(1, h - 1).wait_recv()
            rdma_a(1, h).start()
            rdma_b(1, h - 1).wait_recv()
            rdma_b(1, h).start()
            for j in range(SUB):
                rdma_a(j, h - 1).wait_send()
                rdma_b(j, h - 1).wait_send()
            if h <= 5:
                pl.semaphore_signal(
                    credit_a, inc=1,
                    device_id=(left,), device_id_type=pl.DeviceIdType.MESH,
                )
                pl.semaphore_signal(
                    credit_b, inc=1,
                    device_id=(right,), device_id_type=pl.DeviceIdType.MESH,
                )
            gemm_store(h)

        def gemm_store_sub(buf, origin_row, j):
            s = (N_DEV - 1) % DEPTH
            y = jnp.dot(buf[s, pl.ds(j * sub, sub), :], w_bf[...],
                        preferred_element_type=jnp.float32)
            out_ref[pl.ds(origin_row + j * sub, sub), :] = (
                jnp.maximum(y, 0.0))

        origin_a = (my - (N_DEV - 1)) % N_DEV
        origin_b = (my + (N_DEV - 1)) % N_DEV
        for j in range(SUB):
            rdma_a(j, N_DEV - 2).wait_recv()
            rdma_b(j, N_DEV - 2).wait_recv()
            gemm_store_sub(buf_a, origin_a * m_per, j)
            gemm_store_sub(buf_b, origin_b * m_per + half, j)
        for j in range(SUB):
            rdma_a(j, N_DEV - 2).wait_send()
            rdma_b(j, N_DEV - 2).wait_send()

    return pl.pallas_call(
        body,
        out_shape=jax.ShapeDtypeStruct((N_DEV * m_per, n_per), jnp.float32),
        in_specs=[
            pl.BlockSpec(memory_space=pltpu.VMEM),
            pl.BlockSpec(memory_space=pltpu.VMEM),
        ],
        out_specs=pl.BlockSpec(memory_space=pltpu.VMEM),
        scratch_shapes=[
            pltpu.VMEM((DEPTH, half, k), jnp.bfloat16),
            pltpu.VMEM((DEPTH, half, k), jnp.bfloat16),
            pltpu.VMEM((k, n_per), jnp.bfloat16),
            pltpu.SemaphoreType.DMA((SUB, N_DEV - 1)),
            pltpu.SemaphoreType.DMA((SUB, N_DEV - 1)),
            pltpu.SemaphoreType.DMA((SUB, N_DEV - 1)),
            pltpu.SemaphoreType.DMA((SUB, N_DEV - 1)),
            pltpu.SemaphoreType.REGULAR,
            pltpu.SemaphoreType.REGULAR,
        ],
        compiler_params=pltpu.CompilerParams(collective_id=0),
    )(x, w_mat)
